# baseline (speedup 1.0000x reference)
import functools

import jax
import jax.numpy as jnp
from jax import lax
from jax.experimental import pallas as pl
from jax.experimental.pallas import tpu as pltpu
from jax.experimental.pallas import tpu_sc as plsc

N = 10000
E = 320000
F_IN = 128
H = 64
NG = 64
EPS = 1e-5

NC = 2
NS = 16
NW = NC * NS
B = 128
G = 4
C = 20
E_PAD = NW * C * G * B
ROWS_PER = 632
N_PAD = NS * ROWS_PER

_mesh = plsc.VectorSubcoreMesh(core_axis_name="c", subcore_axis_name="s")


@functools.partial(
    pl.kernel,
    mesh=_mesh,
    compiler_params=pltpu.CompilerParams(use_tc_tiling_on_sc=False),
    out_type=jax.ShapeDtypeStruct((NC * N_PAD, H), jnp.float32),
    scratch_types=[
        pltpu.VMEM((C, G * B), jnp.int32),
        pltpu.VMEM((C, G * B), jnp.int32),
        pltpu.VMEM((G * B, H), jnp.float32),
        pltpu.VMEM((G * B, H), jnp.float32),
        pltpu.VMEM_SHARED((N_PAD, H), jnp.float32),
        pltpu.SemaphoreType.DMA,
        pltpu.SemaphoreType.DMA,
    ],
)
def _edge_agg(xt_hbm, src_hbm, dst_hbm, zero_hbm, out_hbm,
              src_v, dst_v, b0, b1, agg_sh, sg0, sg1):
    cid = lax.axis_index("c")
    sid = lax.axis_index("s")
    wid = sid * NC + cid

    pltpu.sync_copy(zero_hbm, agg_sh.at[pl.ds(sid * ROWS_PER, ROWS_PER)])
    pltpu.sync_copy(src_hbm.at[wid], src_v)
    pltpu.sync_copy(dst_hbm.at[wid], dst_v)
    plsc.subcore_barrier()

    bufs = (b0, b1)
    sgs = (sg0, sg1)

    def g(j, u):
        pltpu.async_copy(xt_hbm.at[src_v.at[j]], bufs[u], sgs[u])

    def gw(j, u):
        pltpu.make_async_copy(xt_hbm.at[src_v.at[j]], bufs[u], sgs[u]).wait()

    def sc(j, u):
        pltpu.sync_copy(bufs[u], agg_sh.at[dst_v.at[j]], add=True)

    g(0, 0)

    def body(i, carry):
        j = 2 * i
        g(j + 1, 1)
        gw(j, 0)
        sc(j, 0)
        g(j + 2, 0)
        gw(j + 1, 1)
        sc(j + 1, 1)
        return carry

    lax.fori_loop(0, C // 2 - 1, body, 0)
    g(C - 1, 1)
    gw(C - 2, 0)
    sc(C - 2, 0)
    gw(C - 1, 1)
    sc(C - 1, 1)

    plsc.subcore_barrier()
    pltpu.sync_copy(
        agg_sh.at[pl.ds(sid * ROWS_PER, ROWS_PER)],
        out_hbm.at[pl.ds(cid * N_PAD + sid * ROWS_PER, ROWS_PER)],
    )


def _bn_rows(h, g, b):
    mu = jnp.mean(h, axis=0, keepdims=True)
    var = jnp.mean((h - mu) ** 2, axis=0, keepdims=True)
    return (h - mu) * lax.rsqrt(var + EPS) * g + b


def _tc1_body(x_ref, wi_ref, bi_ref, gi_ref, bei_ref, wrel_ref, wroot_ref,
              bg_ref, xt_ref, rt_ref):
    h = jnp.dot(x_ref[...], wi_ref[...], preferred_element_type=jnp.float32)
    h = _bn_rows(h + bi_ref[...], gi_ref[...], bei_ref[...])
    h = jnp.maximum(h, 0.0)
    xt_ref[...] = jnp.dot(h, wrel_ref[...], preferred_element_type=jnp.float32)
    rt_ref[...] = (
        jnp.dot(h, wroot_ref[...], preferred_element_type=jnp.float32)
        + bg_ref[...]
    )


_tc1 = pl.pallas_call(
    _tc1_body,
    out_shape=(
        jax.ShapeDtypeStruct((N, H), jnp.float32),
        jax.ShapeDtypeStruct((N, H), jnp.float32),
    ),
)


def _tc2_body(a0_ref, a1_ref, rt_ref, wrel_ref, wroot_ref, bg_ref,
              xt_ref, rtn_ref):
    h = jnp.maximum(a0_ref[...] + a1_ref[...] + rt_ref[...], 0.0)
    xt_ref[...] = jnp.dot(h, wrel_ref[...], preferred_element_type=jnp.float32)
    rtn_ref[...] = (
        jnp.dot(h, wroot_ref[...], preferred_element_type=jnp.float32)
        + bg_ref[...]
    )


_tc2 = pl.pallas_call(
    _tc2_body,
    out_shape=(
        jax.ShapeDtypeStruct((N, H), jnp.float32),
        jax.ShapeDtypeStruct((N, H), jnp.float32),
    ),
)


def _tc3_body(a0_ref, a1_ref, rt_ref, x_ref, hom_ref, wgx_ref, wgh_ref,
              wghom_ref, bg_ref, gg_ref, beg_ref, wtr_ref, btr_ref, gtr_ref,
              betr_ref, batch_ref, out_ref):
    h = jnp.maximum(a0_ref[...] + a1_ref[...] + rt_ref[...], 0.0)
    gate = (
        jnp.sum(x_ref[...] * wgx_ref[...], axis=1, keepdims=True)
        + jnp.sum(h * wgh_ref[...], axis=1, keepdims=True)
        + jnp.sum(hom_ref[...] * wghom_ref[...], axis=1, keepdims=True)
        + bg_ref[...]
    )
    gate = _bn_rows(gate, gg_ref[...], beg_ref[...])
    tr = jnp.sum(h * wtr_ref[...], axis=1, keepdims=True) + btr_ref[...]
    tr = _bn_rows(tr, gtr_ref[...], betr_ref[...])
    prod = jax.nn.sigmoid(gate) * tr
    seg = lax.broadcasted_iota(jnp.int32, (N, NG), 1)
    m = batch_ref[...] == seg
    out_ref[...] = jnp.sum(jnp.where(m, prod, 0.0), axis=0, keepdims=True)


_tc3 = pl.pallas_call(
    _tc3_body,
    out_shape=jax.ShapeDtypeStruct((1, NG), jnp.float32),
)


def _prep_edges(ei):
    pad = E_PAD - E
    pad_i = jnp.arange(pad, dtype=jnp.int32)
    pad_src = (pad_i * 61) % N
    pad_dst = N + pad_i % (N_PAD - N)
    src = jnp.concatenate([ei[0], pad_src]).reshape(NW, C, G * B)
    dst = jnp.concatenate([ei[1], pad_dst]).reshape(NW, C, G * B)
    return src, dst


def kernel(x, graph_hom, W_init, b_init, g_init, be_init, W_rel0, W_root0,
           b_gcn0, W_rel1, W_root1, b_gcn1, W_gate, b_gate, g_gate, be_gate,
           W_tr, b_tr, g_tr, be_tr, edge_index, edge_attr, fa_edge_index,
           fa_edge_attr, batch):
    src0, dst0 = _prep_edges(edge_index)
    src1, dst1 = _prep_edges(fa_edge_index)
    zeros = jnp.zeros((ROWS_PER, H), jnp.float32)

    xt0, rt0 = _tc1(x, W_init, b_init.reshape(1, H), g_init.reshape(1, H),
                    be_init.reshape(1, H), W_rel0[0], W_root0,
                    b_gcn0.reshape(1, H))
    parts0 = _edge_agg(xt0, src0, dst0, zeros)
    xt1, rt1 = _tc2(parts0[:N], parts0[N_PAD:N_PAD + N], rt0, W_rel1[0],
                    W_root1, b_gcn1.reshape(1, H))
    parts1 = _edge_agg(xt1, src1, dst1, zeros)

    hom_p = jnp.pad(graph_hom, ((0, 0), (0, 1)))
    wgx = W_gate[:F_IN, 0].reshape(1, F_IN)
    wgh = W_gate[F_IN:F_IN + H, 0].reshape(1, H)
    wghom = jnp.pad(W_gate[F_IN + H:, 0], (0, 1)).reshape(1, 32)
    out = _tc3(parts1[:N], parts1[N_PAD:N_PAD + N], rt1, x, hom_p,
               wgx, wgh, wghom,
               b_gate.reshape(1, 1), g_gate.reshape(1, 1),
               be_gate.reshape(1, 1), W_tr.reshape(1, H),
               b_tr.reshape(1, 1), g_tr.reshape(1, 1), be_tr.reshape(1, 1),
               batch.reshape(N, 1))
    return out.reshape(NG, 1)


# speedup vs baseline: 1.0363x; 1.0363x over previous
import functools

import jax
import jax.numpy as jnp
from jax import lax
from jax.experimental import pallas as pl
from jax.experimental.pallas import tpu as pltpu
from jax.experimental.pallas import tpu_sc as plsc

N = 10000
E = 320000
F_IN = 128
H = 64
NG = 64
EPS = 1e-5

NC = 2
NS = 16
NW = NC * NS
B = 128
C = 80
E_PAD = NW * C * B
ROWS_PER = 632
N_PAD = NS * ROWS_PER

_mesh = plsc.VectorSubcoreMesh(core_axis_name="c", subcore_axis_name="s")


@functools.partial(
    pl.kernel,
    mesh=_mesh,
    compiler_params=pltpu.CompilerParams(use_tc_tiling_on_sc=False),
    out_type=jax.ShapeDtypeStruct((NC * N_PAD, H), jnp.float32),
    scratch_types=[
        pltpu.VMEM((C, B), jnp.int32),
        pltpu.VMEM((C, B), jnp.int32),
        pltpu.VMEM((B, H), jnp.float32),
        pltpu.VMEM((B, H), jnp.float32),
        pltpu.VMEM((B, H), jnp.float32),
        pltpu.VMEM((B, H), jnp.float32),
        pltpu.VMEM_SHARED((N_PAD, H), jnp.float32),
        pltpu.SemaphoreType.DMA,
        pltpu.SemaphoreType.DMA,
        pltpu.SemaphoreType.DMA,
        pltpu.SemaphoreType.DMA,
    ],
)
def _edge_agg(xt_hbm, src_hbm, dst_hbm, zero_hbm, out_hbm,
              src_v, dst_v, b0, b1, b2, b3, agg_sh, sg0, sg1, sg2, sg3):
    cid = lax.axis_index("c")
    sid = lax.axis_index("s")
    wid = sid * NC + cid

    pltpu.sync_copy(zero_hbm, agg_sh.at[pl.ds(sid * ROWS_PER, ROWS_PER)])
    pltpu.sync_copy(src_hbm.at[wid], src_v)
    pltpu.sync_copy(dst_hbm.at[wid], dst_v)
    plsc.subcore_barrier()

    bufs = (b0, b1, b2, b3)
    sgs = (sg0, sg1, sg2, sg3)

    def g(j, u):
        pltpu.async_copy(xt_hbm.at[src_v.at[j]], bufs[u], sgs[u])

    def gw(j, u):
        pltpu.make_async_copy(xt_hbm.at[src_v.at[j]], bufs[u], sgs[u]).wait()

    def sc(j, u):
        pltpu.sync_copy(bufs[u], agg_sh.at[dst_v.at[j]], add=True)

    g(0, 0)
    g(1, 1)
    g(2, 2)
    gw(0, 0); sc(0, 0); g(3, 3)
    gw(1, 1); sc(1, 1); g(4, 0)
    gw(2, 2); sc(2, 2); g(5, 1)
    gw(3, 3); sc(3, 3); g(6, 2)

    def body(i, carry):
        j0 = 4 * i
        for u in range(4):
            j = j0 + u
            gw(j, u)
            sc(j, u)
            g(j + 3, (u + 3) % 4)
        return carry

    lax.fori_loop(1, C // 4 - 1, body, 0)
    gw(C - 4, 0); sc(C - 4, 0); g(C - 1, 3)
    gw(C - 3, 1); sc(C - 3, 1)
    gw(C - 2, 2); sc(C - 2, 2)
    gw(C - 1, 3); sc(C - 1, 3)

    plsc.subcore_barrier()
    pltpu.sync_copy(
        agg_sh.at[pl.ds(sid * ROWS_PER, ROWS_PER)],
        out_hbm.at[pl.ds(cid * N_PAD + sid * ROWS_PER, ROWS_PER)],
    )


def _bn_rows(h, g, b):
    mu = jnp.mean(h, axis=0, keepdims=True)
    var = jnp.mean((h - mu) ** 2, axis=0, keepdims=True)
    return (h - mu) * lax.rsqrt(var + EPS) * g + b


def _tc1_body(x_ref, wi_ref, bi_ref, gi_ref, bei_ref, wrel_ref, wroot_ref,
              bg_ref, xt_ref, rt_ref):
    h = jnp.dot(x_ref[...], wi_ref[...], preferred_element_type=jnp.float32)
    h = _bn_rows(h + bi_ref[...], gi_ref[...], bei_ref[...])
    h = jnp.maximum(h, 0.0)
    xt_ref[...] = jnp.dot(h, wrel_ref[...], preferred_element_type=jnp.float32)
    rt_ref[...] = (
        jnp.dot(h, wroot_ref[...], preferred_element_type=jnp.float32)
        + bg_ref[...]
    )


_tc1 = pl.pallas_call(
    _tc1_body,
    out_shape=(
        jax.ShapeDtypeStruct((N, H), jnp.float32),
        jax.ShapeDtypeStruct((N, H), jnp.float32),
    ),
)


def _tc2_body(a0_ref, a1_ref, rt_ref, wrel_ref, wroot_ref, bg_ref,
              xt_ref, rtn_ref):
    h = jnp.maximum(a0_ref[...] + a1_ref[...] + rt_ref[...], 0.0)
    xt_ref[...] = jnp.dot(h, wrel_ref[...], preferred_element_type=jnp.float32)
    rtn_ref[...] = (
        jnp.dot(h, wroot_ref[...], preferred_element_type=jnp.float32)
        + bg_ref[...]
    )


_tc2 = pl.pallas_call(
    _tc2_body,
    out_shape=(
        jax.ShapeDtypeStruct((N, H), jnp.float32),
        jax.ShapeDtypeStruct((N, H), jnp.float32),
    ),
)


def _tcg_body(x_ref, hom_ref, wgx_ref, wghom_ref, bg_ref, gp_ref):
    gp_ref[...] = (
        jnp.sum(x_ref[...] * wgx_ref[...], axis=1, keepdims=True)
        + jnp.sum(hom_ref[...] * wghom_ref[...], axis=1, keepdims=True)
        + bg_ref[...]
    )


_tcg = pl.pallas_call(
    _tcg_body,
    out_shape=jax.ShapeDtypeStruct((N, 1), jnp.float32),
)


def _tc3_body(a0_ref, a1_ref, rt_ref, gp_ref, wgh_ref,
              gg_ref, beg_ref, wtr_ref, btr_ref, gtr_ref,
              betr_ref, batch_ref, out_ref):
    h = jnp.maximum(a0_ref[...] + a1_ref[...] + rt_ref[...], 0.0)
    gate = gp_ref[...] + jnp.sum(h * wgh_ref[...], axis=1, keepdims=True)
    gate = _bn_rows(gate, gg_ref[...], beg_ref[...])
    tr = jnp.sum(h * wtr_ref[...], axis=1, keepdims=True) + btr_ref[...]
    tr = _bn_rows(tr, gtr_ref[...], betr_ref[...])
    prod = jax.nn.sigmoid(gate) * tr
    seg = lax.broadcasted_iota(jnp.int32, (N, NG), 1)
    m = batch_ref[...] == seg
    out_ref[...] = jnp.sum(jnp.where(m, prod, 0.0), axis=0, keepdims=True)


_tc3 = pl.pallas_call(
    _tc3_body,
    out_shape=jax.ShapeDtypeStruct((1, NG), jnp.float32),
)


def _prep_edges(ei):
    pad = E_PAD - E
    pad_i = jnp.arange(pad, dtype=jnp.int32)
    pad_src = (pad_i * 61) % N
    pad_dst = N + pad_i % (N_PAD - N)
    src = jnp.concatenate([ei[0], pad_src]).reshape(NW, C, B)
    dst = jnp.concatenate([ei[1], pad_dst]).reshape(NW, C, B)
    return src, dst


def kernel(x, graph_hom, W_init, b_init, g_init, be_init, W_rel0, W_root0,
           b_gcn0, W_rel1, W_root1, b_gcn1, W_gate, b_gate, g_gate, be_gate,
           W_tr, b_tr, g_tr, be_tr, edge_index, edge_attr, fa_edge_index,
           fa_edge_attr, batch):
    src0, dst0 = _prep_edges(edge_index)
    src1, dst1 = _prep_edges(fa_edge_index)
    zeros = jnp.zeros((ROWS_PER, H), jnp.float32)

    xt0, rt0 = _tc1(x, W_init, b_init.reshape(1, H), g_init.reshape(1, H),
                    be_init.reshape(1, H), W_rel0[0], W_root0,
                    b_gcn0.reshape(1, H))
    parts0 = _edge_agg(xt0, src0, dst0, zeros)
    xt1, rt1 = _tc2(parts0[:N], parts0[N_PAD:N_PAD + N], rt0, W_rel1[0],
                    W_root1, b_gcn1.reshape(1, H))
    parts1 = _edge_agg(xt1, src1, dst1, zeros)

    hom_p = jnp.pad(graph_hom, ((0, 0), (0, 1)))
    wgx = W_gate[:F_IN, 0].reshape(1, F_IN)
    wgh = W_gate[F_IN:F_IN + H, 0].reshape(1, H)
    wghom = jnp.pad(W_gate[F_IN + H:, 0], (0, 1)).reshape(1, 32)
    gp = _tcg(x, hom_p, wgx, wghom, b_gate.reshape(1, 1))
    out = _tc3(parts1[:N], parts1[N_PAD:N_PAD + N], rt1, gp, wgh,
               g_gate.reshape(1, 1),
               be_gate.reshape(1, 1), W_tr.reshape(1, H),
               b_tr.reshape(1, 1), g_tr.reshape(1, 1), be_tr.reshape(1, 1),
               batch.reshape(N, 1))
    return out.reshape(NG, 1)


# speedup vs baseline: 1.2100x; 1.1676x over previous
import functools

import jax
import jax.numpy as jnp
from jax import lax
from jax.experimental import pallas as pl
from jax.experimental.pallas import tpu as pltpu
from jax.experimental.pallas import tpu_sc as plsc

N = 10000
E = 320000
F_IN = 128
H = 64
NG = 64
EPS = 1e-5

NC = 2
NS = 16
NW = NC * NS
B = 128
C = 80
E_PAD = NW * C * B
ROWS_PER = 632
N_PAD = NS * ROWS_PER

_mesh = plsc.VectorSubcoreMesh(core_axis_name="c", subcore_axis_name="s")


@functools.partial(
    pl.kernel,
    mesh=_mesh,
    compiler_params=pltpu.CompilerParams(use_tc_tiling_on_sc=False),
    out_type=jax.ShapeDtypeStruct((N_PAD, NC * H), jnp.float32),
    scratch_types=[
        pltpu.VMEM((C, B), jnp.int32),
        pltpu.VMEM((C, B), jnp.int32),
        pltpu.VMEM((B, H), jnp.float32),
        pltpu.VMEM((B, H), jnp.float32),
        pltpu.VMEM((B, H), jnp.float32),
        pltpu.VMEM((B, H), jnp.float32),
        pltpu.VMEM_SHARED((N_PAD, H), jnp.float32),
        pltpu.SemaphoreType.DMA,
        pltpu.SemaphoreType.DMA,
        pltpu.SemaphoreType.DMA,
        pltpu.SemaphoreType.DMA,
    ],
)
def _edge_agg(xt_hbm, src_hbm, dst_hbm, zero_hbm, out_hbm,
              src_v, dst_v, b0, b1, b2, b3, agg_sh, sg0, sg1, sg2, sg3):
    cid = lax.axis_index("c")
    sid = lax.axis_index("s")
    wid = sid * NC + cid

    pltpu.sync_copy(zero_hbm, agg_sh.at[pl.ds(sid * ROWS_PER, ROWS_PER)])
    pltpu.sync_copy(src_hbm.at[wid], src_v)
    pltpu.sync_copy(dst_hbm.at[wid], dst_v)
    plsc.subcore_barrier()

    bufs = (b0, b1, b2, b3)
    sgs = (sg0, sg1, sg2, sg3)

    def g(j, u):
        pltpu.async_copy(xt_hbm.at[src_v.at[j]], bufs[u], sgs[u])

    def gw(j, u):
        pltpu.make_async_copy(xt_hbm.at[src_v.at[j]], bufs[u], sgs[u]).wait()

    def sc(j, u):
        pltpu.sync_copy(bufs[u], agg_sh.at[dst_v.at[j]], add=True)

    g(0, 0)
    g(1, 1)
    g(2, 2)
    gw(0, 0); sc(0, 0); g(3, 3)
    gw(1, 1); sc(1, 1); g(4, 0)
    gw(2, 2); sc(2, 2); g(5, 1)
    gw(3, 3); sc(3, 3); g(6, 2)

    def body(i, carry):
        j0 = 4 * i
        for u in range(4):
            j = j0 + u
            gw(j, u)
            sc(j, u)
            g(j + 3, (u + 3) % 4)
        return carry

    lax.fori_loop(1, C // 4 - 1, body, 0)
    gw(C - 4, 0); sc(C - 4, 0); g(C - 1, 3)
    gw(C - 3, 1); sc(C - 3, 1)
    gw(C - 2, 2); sc(C - 2, 2)
    gw(C - 1, 3); sc(C - 1, 3)

    plsc.subcore_barrier()
    pltpu.sync_copy(
        agg_sh.at[pl.ds(sid * ROWS_PER, ROWS_PER)],
        out_hbm.at[pl.ds(sid * ROWS_PER, ROWS_PER), pl.ds(cid * H, H)],
    )


def _bn_rows(h, g, b):
    mu = jnp.mean(h, axis=0, keepdims=True)
    var = jnp.mean((h - mu) ** 2, axis=0, keepdims=True)
    return (h - mu) * lax.rsqrt(var + EPS) * g + b


def _tc1_body(x_ref, wi_ref, bi_ref, gi_ref, bei_ref, wrel_ref, wroot_ref,
              bg_ref, xt_ref, rt_ref):
    h = jnp.dot(x_ref[...], wi_ref[...], preferred_element_type=jnp.float32)
    h = _bn_rows(h + bi_ref[...], gi_ref[...], bei_ref[...])
    h = jnp.maximum(h, 0.0)
    xt_ref[...] = jnp.dot(h, wrel_ref[...], preferred_element_type=jnp.float32)
    rt_ref[...] = (
        jnp.dot(h, wroot_ref[...], preferred_element_type=jnp.float32)
        + bg_ref[...]
    )


_tc1 = pl.pallas_call(
    _tc1_body,
    out_shape=(
        jax.ShapeDtypeStruct((N, H), jnp.float32),
        jax.ShapeDtypeStruct((N, H), jnp.float32),
    ),
)


def _tc2_body(parts_ref, rt_ref, wrel_ref, wroot_ref, bg_ref,
              xt_ref, rtn_ref):
    h = jnp.maximum(parts_ref[:N, :H] + parts_ref[:N, H:] + rt_ref[...], 0.0)
    xt_ref[...] = jnp.dot(h, wrel_ref[...], preferred_element_type=jnp.float32)
    rtn_ref[...] = (
        jnp.dot(h, wroot_ref[...], preferred_element_type=jnp.float32)
        + bg_ref[...]
    )


_tc2 = pl.pallas_call(
    _tc2_body,
    out_shape=(
        jax.ShapeDtypeStruct((N, H), jnp.float32),
        jax.ShapeDtypeStruct((N, H), jnp.float32),
    ),
)


def _tcg_body(x_ref, hom_ref, wgx_ref, wghom_ref, bg_ref, gp_ref):
    gp_ref[...] = (
        jnp.sum(x_ref[...] * wgx_ref[...], axis=1, keepdims=True)
        + jnp.sum(hom_ref[...] * wghom_ref[...], axis=1, keepdims=True)
        + bg_ref[...]
    )


_tcg = pl.pallas_call(
    _tcg_body,
    out_shape=jax.ShapeDtypeStruct((N, 1), jnp.float32),
)


def _tc3_body(parts_ref, rt_ref, gp_ref, wgh_ref,
              gg_ref, beg_ref, wtr_ref, btr_ref, gtr_ref,
              betr_ref, batch_ref, out_ref):
    h = jnp.maximum(parts_ref[:N, :H] + parts_ref[:N, H:] + rt_ref[...], 0.0)
    gate = gp_ref[...] + jnp.sum(h * wgh_ref[...], axis=1, keepdims=True)
    gate = _bn_rows(gate, gg_ref[...], beg_ref[...])
    tr = jnp.sum(h * wtr_ref[...], axis=1, keepdims=True) + btr_ref[...]
    tr = _bn_rows(tr, gtr_ref[...], betr_ref[...])
    prod = jax.nn.sigmoid(gate) * tr
    seg = lax.broadcasted_iota(jnp.int32, (N, NG), 1)
    m = batch_ref[...] == seg
    out_ref[...] = jnp.sum(jnp.where(m, prod, 0.0), axis=0, keepdims=True)


_tc3 = pl.pallas_call(
    _tc3_body,
    out_shape=jax.ShapeDtypeStruct((1, NG), jnp.float32),
)


def _prep_edges(ei):
    pad = E_PAD - E
    pad_i = jnp.arange(pad, dtype=jnp.int32)
    pad_src = (pad_i * 61) % N
    pad_dst = N + pad_i % (N_PAD - N)
    src = jnp.concatenate([ei[0], pad_src]).reshape(NW, C, B)
    dst = jnp.concatenate([ei[1], pad_dst]).reshape(NW, C, B)
    return src, dst


def kernel(x, graph_hom, W_init, b_init, g_init, be_init, W_rel0, W_root0,
           b_gcn0, W_rel1, W_root1, b_gcn1, W_gate, b_gate, g_gate, be_gate,
           W_tr, b_tr, g_tr, be_tr, edge_index, edge_attr, fa_edge_index,
           fa_edge_attr, batch):
    src0, dst0 = _prep_edges(edge_index)
    src1, dst1 = _prep_edges(fa_edge_index)
    zeros = jnp.zeros((ROWS_PER, H), jnp.float32)

    xt0, rt0 = _tc1(x, W_init, b_init.reshape(1, H), g_init.reshape(1, H),
                    be_init.reshape(1, H), W_rel0[0], W_root0,
                    b_gcn0.reshape(1, H))
    parts0 = _edge_agg(xt0, src0, dst0, zeros)
    xt1, rt1 = _tc2(parts0, rt0, W_rel1[0],
                    W_root1, b_gcn1.reshape(1, H))
    parts1 = _edge_agg(xt1, src1, dst1, zeros)

    hom_p = jnp.pad(graph_hom, ((0, 0), (0, 1)))
    wgx = W_gate[:F_IN, 0].reshape(1, F_IN)
    wgh = W_gate[F_IN:F_IN + H, 0].reshape(1, H)
    wghom = jnp.pad(W_gate[F_IN + H:, 0], (0, 1)).reshape(1, 32)
    gp = _tcg(x, hom_p, wgx, wghom, b_gate.reshape(1, 1))
    out = _tc3(parts1, rt1, gp, wgh,
               g_gate.reshape(1, 1),
               be_gate.reshape(1, 1), W_tr.reshape(1, H),
               b_tr.reshape(1, 1), g_tr.reshape(1, 1), be_tr.reshape(1, 1),
               batch.reshape(N, 1))
    return out.reshape(NG, 1)
